# baseline (speedup 1.0000x reference)
import functools
import jax
import jax.numpy as jnp
from jax.experimental import pallas as pl

_B = 2
_S = 256
_H = 4
_D = 64
_V = 130
_TI = 128


def _tc_body(tk2_ref, tv2_ref, tk1_ref, tv1_ref, ide_ref, outk_ref, outv_ref):
    tk2 = tk2_ref[...]
    tv2 = tv2_ref[...]
    tk1 = jnp.broadcast_to(tk1_ref[...], (_D, _H * _D))
    tv1 = jnp.broadcast_to(tv1_ref[...], (_D, _H * _D))
    zero = jnp.zeros((_D, _H * _D), jnp.float32)
    for t in range(_TI):
        ids_row = ide_ref[0, t]
        i2 = jnp.broadcast_to(jnp.maximum(ids_row - 2, 0), (_D, _H * _D))
        big = jnp.broadcast_to(ids_row >= 2, (_D, _H * _D))
        one = jnp.broadcast_to(ids_row == 1, (_D, _H * _D))
        gk = jnp.take_along_axis(tk2, i2, axis=1)
        gv = jnp.take_along_axis(tv2, i2, axis=1)
        outk_ref[0, t] = jnp.where(big, gk, jnp.where(one, tk1, zero))
        outv_ref[0, t] = jnp.where(big, gv, jnp.where(one, tv1, zero))


def kernel(inputs, brother_table, relation_type, num_heads):
    del relation_type, num_heads
    tk = brother_table[:, :_D].T
    tv = brother_table[:, _D:].T
    tk2, tv2 = tk[:, 2:], tv[:, 2:]
    tk1, tv1 = tk[:, 1:2], tv[:, 1:2]

    ide = jnp.broadcast_to(
        inputs.reshape(_B, _S, _S, 1), (_B, _S, _S, _H)
    ).reshape(_B * _S * _S // (_TI * _D), _TI, 1, _H * _D)

    grid = (_B * _S * _S // (_TI * _D),)

    def ide_map(g):
        return (g, 0, 0, 0)

    def out_map(g):
        return (g // (_S // _TI), g % (_S // _TI), 0, 0)

    f = pl.pallas_call(
        _tc_body,
        grid=grid,
        in_specs=[
            pl.BlockSpec((_D, 128), lambda g: (0, 0)),
            pl.BlockSpec((_D, 128), lambda g: (0, 0)),
            pl.BlockSpec((_D, 1), lambda g: (0, 0)),
            pl.BlockSpec((_D, 1), lambda g: (0, 0)),
            pl.BlockSpec((1, _TI, 1, _H * _D), ide_map),
        ],
        out_specs=[
            pl.BlockSpec((1, _TI, _D, _H * _D), out_map),
            pl.BlockSpec((1, _TI, _D, _H * _D), out_map),
        ],
        out_shape=[
            jax.ShapeDtypeStruct((_B * _H, _S, _D, _S), jnp.float32),
            jax.ShapeDtypeStruct((_B * _H, _S, _D, _S), jnp.float32),
        ],
    )
    outk, outv = f(tk2, tv2, tk1, tv1, ide)
    return (
        jnp.transpose(outk, (0, 1, 3, 2)),
        jnp.transpose(outv, (0, 1, 3, 2)),
    )


# speedup vs baseline: 1.0043x; 1.0043x over previous
import functools
import jax
import jax.numpy as jnp
from jax.experimental import pallas as pl
from jax.experimental.pallas import tpu as pltpu

_B = 2
_S = 256
_H = 4
_D = 64
_V = 130
_TI = 64


def _tc_body(tk2_ref, tv2_ref, tk1_ref, tv1_ref, ide_ref, outk_ref, outv_ref):
    tk2 = tk2_ref[...]
    tv2 = tv2_ref[...]
    tk1 = jnp.broadcast_to(tk1_ref[...], (_D, _H * _D))
    tv1 = jnp.broadcast_to(tv1_ref[...], (_D, _H * _D))
    zero = jnp.zeros((_D, _H * _D), jnp.float32)
    for t in range(_TI):
        ids_row = ide_ref[0, t]
        i2 = jnp.broadcast_to(jnp.maximum(ids_row - 2, 0), (_D, _H * _D))
        big = jnp.broadcast_to(ids_row >= 2, (_D, _H * _D))
        one = jnp.broadcast_to(ids_row == 1, (_D, _H * _D))
        gk = jnp.take_along_axis(tk2, i2, axis=1)
        gv = jnp.take_along_axis(tv2, i2, axis=1)
        outk_ref[0, t] = jnp.where(big, gk, jnp.where(one, tk1, zero))
        outv_ref[0, t] = jnp.where(big, gv, jnp.where(one, tv1, zero))


def kernel(inputs, brother_table, relation_type, num_heads):
    del relation_type, num_heads
    tk = brother_table[:, :_D].T
    tv = brother_table[:, _D:].T
    tk2, tv2 = tk[:, 2:], tv[:, 2:]
    tk1, tv1 = tk[:, 1:2], tv[:, 1:2]

    ide = jnp.broadcast_to(
        inputs.reshape(_B, _S, _S, 1), (_B, _S, _S, _H)
    ).reshape(_B * _S * _S // (_TI * _D), _TI, 1, _H * _D)

    grid = (_B * _S * _S // (_TI * _D),)

    def ide_map(g):
        return (g, 0, 0, 0)

    def out_map(g):
        return (g // (_S // _TI), g % (_S // _TI), 0, 0)

    f = pl.pallas_call(
        _tc_body,
        grid=grid,
        compiler_params=pltpu.CompilerParams(
            dimension_semantics=("parallel",)),
        in_specs=[
            pl.BlockSpec((_D, 128), lambda g: (0, 0)),
            pl.BlockSpec((_D, 128), lambda g: (0, 0)),
            pl.BlockSpec((_D, 1), lambda g: (0, 0)),
            pl.BlockSpec((_D, 1), lambda g: (0, 0)),
            pl.BlockSpec((1, _TI, 1, _H * _D), ide_map),
        ],
        out_specs=[
            pl.BlockSpec((1, _TI, _D, _H * _D), out_map),
            pl.BlockSpec((1, _TI, _D, _H * _D), out_map),
        ],
        out_shape=[
            jax.ShapeDtypeStruct((_B * _H, _S, _D, _S), jnp.float32),
            jax.ShapeDtypeStruct((_B * _H, _S, _D, _S), jnp.float32),
        ],
    )
    outk, outv = f(tk2, tv2, tk1, tv1, ide)
    return (
        jnp.transpose(outk, (0, 1, 3, 2)),
        jnp.transpose(outv, (0, 1, 3, 2)),
    )
